# Initial kernel scaffold; baseline (speedup 1.0000x reference)
import jax
import jax.numpy as jnp
from jax.experimental import pallas as pl


def kernel(cache_k, cache_v, new_k, new_v, position_ids):
    raise NotImplementedError("write your pallas kernel here")


# speedup vs baseline: 1.0116x; 1.0116x over previous
import functools

import jax
import jax.numpy as jnp
from jax import lax
from jax.experimental import pallas as pl
from jax.experimental.pallas import tpu as pltpu
from jax.experimental.pallas import tpu_sc as plsc

B, H, S, D, T = 8, 16, 2048, 128, 16

NC, NS = 2, 16
NW = NC * NS
ROWS = B * H * T
RPW = ROWS // NW
GPW = RPW // T

_mesh = plsc.VectorSubcoreMesh(core_axis_name="c", subcore_axis_name="s")


@functools.partial(
    pl.kernel,
    out_type=(),
    mesh=_mesh,
    scratch_types=[
        pltpu.VMEM((T,), jnp.int32),
        pltpu.VMEM((RPW,), jnp.int32),
        pltpu.VMEM((RPW, D), jnp.float32),
        pltpu.VMEM((RPW, D), jnp.float32),
        pltpu.SemaphoreType.DMA,
        pltpu.SemaphoreType.DMA,
    ],
)
def _scatter_update(ck_ref, cv_ref, nk_hbm, nv_hbm, pos_hbm,
                    pos_v, idx_v, krows_v, vrows_v, semk, semv):
    wid = lax.axis_index("s") * NC + lax.axis_index("c")
    base = wid * RPW
    pltpu.sync_copy(pos_hbm, pos_v)
    pltpu.sync_copy(nk_hbm.at[pl.ds(base, RPW)], krows_v)
    pltpu.sync_copy(nv_hbm.at[pl.ds(base, RPW)], vrows_v)
    pos = pos_v[...]
    for g in range(GPW):
        bh = wid * GPW + g
        idx_v[pl.ds(g * T, T)] = pos + bh * S
    cpk = pltpu.async_copy(krows_v, ck_ref.at[idx_v], semk)
    cpv = pltpu.async_copy(vrows_v, cv_ref.at[idx_v], semv)
    cpk.wait()
    cpv.wait()


def kernel(cache_k, cache_v, new_k, new_v, position_ids):
    ck = jax.new_ref(cache_k.reshape(B * H * S, D))
    cv = jax.new_ref(cache_v.reshape(B * H * S, D))
    _scatter_update(ck, cv,
                    new_k.reshape(ROWS, D),
                    new_v.reshape(ROWS, D),
                    position_ids.astype(jnp.int32))
    return (ck[...].reshape(B, H, S, D), cv[...].reshape(B, H, S, D))
